# Initial kernel scaffold; baseline (speedup 1.0000x reference)
import jax
import jax.numpy as jnp
from jax.experimental import pallas as pl


def kernel(x, edge_index, W1l, W1r, b1, W2l, W2r, b2, Wout, bout):
    raise NotImplementedError("write your pallas kernel here")


# speedup vs baseline: 3.0669x; 3.0669x over previous
import jax
import jax.numpy as jnp
from jax import lax
from jax.experimental import pallas as pl
from jax.experimental.pallas import tpu as pltpu
from jax.experimental.pallas import tpu_sc as plsc

N_NODES = 10000
N_EDGES = 320000
D = 128

NC = 2
NS = 16
NW = NC * NS

NPAD = 10240
DUMP = 10200
CHUNK = 128
GB = 8
NCHUNK = ((N_EDGES // NW + CHUNK * GB - 1) // (CHUNK * GB)) * GB
NGRP = NCHUNK // GB
EPT = NCHUNK * CHUNK
EPAD = EPT * NW

ROWS_PER_TILE = NPAD // NS


def _make_sc_agg(with_count: bool):
    mesh = plsc.VectorSubcoreMesh(
        core_axis_name="c", subcore_axis_name="s", num_cores=NC, num_subcores=NS
    )
    out_type = [jax.ShapeDtypeStruct((NC, NPAD, D), jnp.float32)]
    if with_count:
        out_type.append(jax.ShapeDtypeStruct((NW, NPAD), jnp.float32))
    scratch = [
        pltpu.VMEM_SHARED((NPAD, D), jnp.float32),
        pltpu.VMEM((GB, CHUNK), jnp.int32),
        pltpu.VMEM((GB, CHUNK), jnp.int32),
        pltpu.VMEM((CHUNK, D), jnp.float32),
        pltpu.VMEM((16, D), jnp.float32),
        pltpu.SemaphoreType.DMA,
    ]
    if with_count:
        scratch.append(pltpu.VMEM((NPAD,), jnp.float32))

    def body(y_hbm, src_hbm, dst_hbm, agg_hbm, *rest):
        if with_count:
            cnt_hbm, acc, src_v, dst_v, rows_v, zbuf, sem, cnt_v = rest
        else:
            acc, src_v, dst_v, rows_v, zbuf, sem = rest
            cnt_hbm = cnt_v = None
        c = lax.axis_index("c")
        s = lax.axis_index("s")
        wid = s * NC + c
        base = s * ROWS_PER_TILE

        zeros16 = jnp.zeros((16,), jnp.float32)

        @pl.loop(0, 16)
        def _zero_zbuf(r):
            for k in range(D // 16):
                zbuf[r, pl.ds(k * 16, 16)] = zeros16

        @pl.loop(0, ROWS_PER_TILE // 16)
        def _zero_acc(i):
            pltpu.sync_copy(zbuf, acc.at[pl.ds(base + i * 16, 16)])

        if with_count:
            @pl.loop(0, NPAD // 16)
            def _zero_cnt(i):
                cnt_v[pl.ds(i * 16, 16)] = zeros16

        plsc.subcore_barrier()

        ones16 = jnp.ones((16,), jnp.float32)

        @pl.loop(0, NGRP)
        def _grp(g):
            pltpu.sync_copy(src_hbm.at[wid].at[pl.ds(g * GB, GB)], src_v)
            pltpu.sync_copy(dst_hbm.at[wid].at[pl.ds(g * GB, GB)], dst_v)

            @pl.loop(0, GB)
            def _edges(j):
                pltpu.async_copy(y_hbm.at[src_v.at[j]], rows_v, sem).wait()
                pltpu.sync_copy(rows_v, acc.at[dst_v.at[j]], add=True)
                if with_count:
                    for k in range(CHUNK // 16):
                        idx = dst_v[j, pl.ds(k * 16, 16)]
                        plsc.addupdate_scatter(cnt_v, [idx], ones16)

        plsc.subcore_barrier()

        @pl.loop(0, ROWS_PER_TILE // CHUNK)
        def _writeout(i):
            sl = pl.ds(base + i * CHUNK, CHUNK)
            pltpu.sync_copy(acc.at[sl], rows_v)
            pltpu.sync_copy(rows_v, agg_hbm.at[c].at[sl])

        if with_count:
            pltpu.sync_copy(cnt_v, cnt_hbm.at[wid])

    return pl.kernel(
        body,
        out_type=tuple(out_type) if with_count else out_type[0],
        mesh=mesh,
        scratch_types=scratch,
        compiler_params=pltpu.CompilerParams(needs_layout_passes=False),
    )


_sc_agg_count = _make_sc_agg(True)
_sc_agg = _make_sc_agg(False)


ROWB = 1280
GRID = NPAD // ROWB


def _tc1_body(x_ref, wl_ref, wr_ref, y_ref, r_ref):
    xb = x_ref[...]
    y_ref[...] = jnp.dot(xb, wl_ref[...], preferred_element_type=jnp.float32)
    r_ref[...] = jnp.dot(xb, wr_ref[...], preferred_element_type=jnp.float32)


_tc1 = pl.pallas_call(
    _tc1_body,
    grid=(GRID,),
    in_specs=[
        pl.BlockSpec((ROWB, D), lambda i: (i, 0)),
        pl.BlockSpec((D, D), lambda i: (0, 0)),
        pl.BlockSpec((D, D), lambda i: (0, 0)),
    ],
    out_specs=[
        pl.BlockSpec((ROWB, D), lambda i: (i, 0)),
        pl.BlockSpec((ROWB, D), lambda i: (i, 0)),
    ],
    out_shape=[
        jax.ShapeDtypeStruct((NPAD, D), jnp.float32),
        jax.ShapeDtypeStruct((NPAD, D), jnp.float32),
    ],
)


def _tc2_body(agg_ref, cnt_ref, r1_ref, b_ref, wl_ref, wr_ref, y2_ref, r2_ref, inv_ref):
    agg = agg_ref[0] + agg_ref[1]
    cnt = jnp.sum(cnt_ref[...], axis=0)[:, None]
    inv = 1.0 / jnp.maximum(cnt, 1.0)
    h = jnp.maximum(agg * inv + r1_ref[...] + b_ref[...], 0.0)
    y2_ref[...] = jnp.dot(h, wl_ref[...], preferred_element_type=jnp.float32)
    r2_ref[...] = jnp.dot(h, wr_ref[...], preferred_element_type=jnp.float32)
    inv_ref[...] = inv


_tc2 = pl.pallas_call(
    _tc2_body,
    grid=(GRID,),
    in_specs=[
        pl.BlockSpec((NC, ROWB, D), lambda i: (0, i, 0)),
        pl.BlockSpec((NW, ROWB), lambda i: (0, i)),
        pl.BlockSpec((ROWB, D), lambda i: (i, 0)),
        pl.BlockSpec((1, D), lambda i: (0, 0)),
        pl.BlockSpec((D, D), lambda i: (0, 0)),
        pl.BlockSpec((D, D), lambda i: (0, 0)),
    ],
    out_specs=[
        pl.BlockSpec((ROWB, D), lambda i: (i, 0)),
        pl.BlockSpec((ROWB, D), lambda i: (i, 0)),
        pl.BlockSpec((ROWB, 1), lambda i: (i, 0)),
    ],
    out_shape=[
        jax.ShapeDtypeStruct((NPAD, D), jnp.float32),
        jax.ShapeDtypeStruct((NPAD, D), jnp.float32),
        jax.ShapeDtypeStruct((NPAD, 1), jnp.float32),
    ],
)


def _tc3_body(agg_ref, inv_ref, r2_ref, b_ref, wo_ref, bo_ref, out_ref):
    agg = agg_ref[0] + agg_ref[1]
    h2 = jnp.maximum(agg * inv_ref[...] + r2_ref[...] + b_ref[...], 0.0)
    out_ref[...] = (
        jnp.dot(h2, wo_ref[...], preferred_element_type=jnp.float32) + bo_ref[...]
    )


_tc3 = pl.pallas_call(
    _tc3_body,
    grid=(GRID,),
    in_specs=[
        pl.BlockSpec((NC, ROWB, D), lambda i: (0, i, 0)),
        pl.BlockSpec((ROWB, 1), lambda i: (i, 0)),
        pl.BlockSpec((ROWB, D), lambda i: (i, 0)),
        pl.BlockSpec((1, D), lambda i: (0, 0)),
        pl.BlockSpec((D, 1), lambda i: (0, 0)),
        pl.BlockSpec((1, 1), lambda i: (0, 0)),
    ],
    out_specs=pl.BlockSpec((ROWB, 1), lambda i: (i, 0)),
    out_shape=jax.ShapeDtypeStruct((NPAD, 1), jnp.float32),
)


def kernel(x, edge_index, W1l, W1r, b1, W2l, W2r, b2, Wout, bout):
    src = edge_index[0].astype(jnp.int32)
    dst = edge_index[1].astype(jnp.int32)
    src3 = (
        jnp.zeros((EPAD,), jnp.int32).at[:N_EDGES].set(src).reshape(NW, NCHUNK, CHUNK)
    )
    dst3 = (
        jnp.full((EPAD,), DUMP, jnp.int32).at[:N_EDGES].set(dst).reshape(NW, NCHUNK, CHUNK)
    )
    x_pad = jnp.pad(x, ((0, NPAD - N_NODES), (0, 0)))
    b1r = b1.reshape(1, D)
    b2r = b2.reshape(1, D)
    boutr = bout.reshape(1, 1)

    y1, r1 = _tc1(x_pad, W1l, W1r)
    agg1, cnt = _sc_agg_count(y1, src3, dst3)
    y2, r2, inv = _tc2(agg1, cnt, r1, b1r, W2l, W2r)
    agg2 = _sc_agg(y2, src3, dst3)
    out = _tc3(agg2, inv, r2, b2r, Wout, boutr)
    return out[:N_NODES]
